# baseline (speedup 1.0000x reference)
import functools

import jax
import jax.numpy as jnp
from jax import lax
from jax.experimental import pallas as pl
from jax.experimental.pallas import tpu as pltpu
from jax.experimental.pallas import tpu_sc as plsc

N_NODES = 10000
N_EDGES = 320000
N_GRAPHS = 64
D = 128

NC = 2
NS = 16
NW = NC * NS

NPAD = 10240
ROWS_PER_TILE = NPAD // NS

CHUNK = 128
NCHUNKS = 80
E_PER_TILE = NCHUNKS * CHUNK
E_PAD = NW * E_PER_TILE - N_EDGES

ROW_BLK = 1000
GRID_N = N_NODES // ROW_BLK

BN_INV = 1.0 / (1.0 + 1e-5) ** 0.5


def _sc_edge_agg(h, src3, dst3):
    mesh = plsc.VectorSubcoreMesh(core_axis_name="c", subcore_axis_name="s")

    @functools.partial(
        pl.kernel,
        mesh=mesh,
        out_type=jax.ShapeDtypeStruct((NC, NPAD, D), jnp.float32),
        scratch_types=[
            pltpu.VMEM((CHUNK,), jnp.int32),
            pltpu.VMEM((CHUNK,), jnp.int32),
            pltpu.VMEM((CHUNK, D), jnp.float32),
            pltpu.VMEM((CHUNK, D), jnp.float32),
            pltpu.VMEM_SHARED((NPAD, D), jnp.float32),
            pltpu.SemaphoreType.DMA,
            pltpu.SemaphoreType.DMA,
            pltpu.SemaphoreType.DMA,
            pltpu.SemaphoreType.DMA,
        ],
    )
    def k(h_hbm, src_hbm, dst_hbm, out_hbm, sidx, didx, rows0, rows1, agg,
          sg0, sg1, ss0, ss1):
        cid = lax.axis_index("c")
        sid = lax.axis_index("s")
        wid = sid * NC + cid
        rows = (rows0, rows1)
        semg = (sg0, sg1)
        sems = (ss0, ss1)

        def zero_row(i, _):
            for j in range(D // 16):
                rows0[i, pl.ds(j * 16, 16)] = jnp.zeros((16,), jnp.float32)
            return 0

        lax.fori_loop(0, CHUNK, zero_row, 0)
        zbase = sid * ROWS_PER_TILE
        for t in range(ROWS_PER_TILE // CHUNK):
            pltpu.sync_copy(rows0, agg.at[pl.ds(zbase + t * CHUNK, CHUNK)])
        plsc.subcore_barrier()

        def body(i, _):
            pltpu.sync_copy(src_hbm.at[wid, i], sidx)
            pltpu.sync_copy(dst_hbm.at[wid, i], didx)
            pltpu.async_copy(h_hbm.at[sidx], rows0, semg[0]).wait()
            pltpu.sync_copy(rows0, agg.at[didx], add=True)
            return 0

        lax.fori_loop(0, NCHUNKS, body, 0)
        plsc.subcore_barrier()

        pltpu.sync_copy(
            agg.at[pl.ds(sid * ROWS_PER_TILE, ROWS_PER_TILE)],
            out_hbm.at[cid, pl.ds(sid * ROWS_PER_TILE, ROWS_PER_TILE)],
        )

    return k(h, src3, dst3)


def _gcn_dense_body(agg_ref, x_ref, W_ref, b_ref, rW_ref, rb_ref, h_ref):
    a = agg_ref[0] + agg_ref[1]
    new = jnp.dot(a, W_ref[...], preferred_element_type=jnp.float32)
    new = jnp.maximum(new + b_ref[...], 0.0)
    r = jnp.dot(x_ref[...], rW_ref[...], preferred_element_type=jnp.float32)
    h_ref[...] = new + jnp.maximum(r + rb_ref[...], 0.0)


def _gcn_dense(aggpair, x, W, b, rW, rb):
    return pl.pallas_call(
        _gcn_dense_body,
        grid=(GRID_N,),
        in_specs=[
            pl.BlockSpec((NC, ROW_BLK, D), lambda i: (0, i, 0)),
            pl.BlockSpec((ROW_BLK, D), lambda i: (i, 0)),
            pl.BlockSpec((D, D), lambda i: (0, 0)),
            pl.BlockSpec((1, D), lambda i: (0, 0)),
            pl.BlockSpec((D, D), lambda i: (0, 0)),
            pl.BlockSpec((1, D), lambda i: (0, 0)),
        ],
        out_specs=pl.BlockSpec((ROW_BLK, D), lambda i: (i, 0)),
        out_shape=jax.ShapeDtypeStruct((N_NODES, D), jnp.float32),
    )(aggpair, x, W, b, rW, rb)


def _head_body(
    agg_ref, h1_ref, W_ref, b_ref, rW_ref, rb_ref, ids_ref, idc_ref,
    wa_ref, ba_ref, Wm1_ref, bm1_ref, Wm2_ref, bm2_ref, Wc_ref, bc_ref,
    add_ref, out_ref, hsum_ref, hmax_ref,
):
    i = pl.program_id(0)
    a = agg_ref[0] + agg_ref[1]
    new = jnp.dot(a, W_ref[...], preferred_element_type=jnp.float32)
    new = jnp.maximum(new + b_ref[...], 0.0)
    r = jnp.dot(h1_ref[...], rW_ref[...], preferred_element_type=jnp.float32)
    h2 = new + jnp.maximum(r + rb_ref[...], 0.0)

    wgt = jax.nn.sigmoid(
        jnp.dot(h2, wa_ref[...], preferred_element_type=jnp.float32)
        + ba_ref[0, 0]
    )

    ids = ids_ref[0, 0, :]
    gi = lax.broadcasted_iota(jnp.int32, (N_GRAPHS, ROW_BLK), 0)
    MT = (gi == ids[None, :]).astype(jnp.float32)
    part_sum = jnp.dot(MT, h2 * wgt, preferred_element_type=jnp.float32)

    @pl.when(i == 0)
    def _():
        hsum_ref[...] = part_sum
        hmax_ref[...] = jnp.full((N_GRAPHS, D), -jnp.inf, jnp.float32)

    @pl.when(i > 0)
    def _():
        hsum_ref[...] += part_sum

    ids_col = idc_ref[...]

    def gmax(g, _):
        hg = jnp.where(ids_col == g, h2, -jnp.inf)
        mx = jnp.max(hg, axis=0)
        hmax_ref[pl.ds(g, 1), :] = jnp.maximum(hmax_ref[pl.ds(g, 1), :], mx[None, :])
        return 0

    lax.fori_loop(jnp.min(ids), jnp.max(ids) + 1, gmax, 0)

    @pl.when(i == GRID_N - 1)
    def _():
        g = jnp.concatenate([hsum_ref[...], hmax_ref[...]], axis=1)
        h1 = jnp.dot(g, Wm1_ref[...], preferred_element_type=jnp.float32)
        h1 = jnp.maximum(h1 + bm1_ref[...], 0.0) * BN_INV
        o = jnp.dot(h1, Wm2_ref[...], preferred_element_type=jnp.float32)
        o = o + bm2_ref[...]
        res = (
            jnp.dot(o, Wc_ref[:D, :], preferred_element_type=jnp.float32)
            + jnp.dot(add_ref[...], Wc_ref[D:, :], preferred_element_type=jnp.float32)
            + bc_ref[0, 0]
        )
        out_ref[...] = res


def _head(aggpair, h1, W, b, rW, rb, ids3, idc, wa, ba, Wm1, bm1, Wm2, bm2,
          Wc, bc, addin):
    n_tasks = Wc.shape[1]
    return pl.pallas_call(
        _head_body,
        grid=(GRID_N,),
        in_specs=[
            pl.BlockSpec((NC, ROW_BLK, D), lambda i: (0, i, 0)),
            pl.BlockSpec((ROW_BLK, D), lambda i: (i, 0)),
            pl.BlockSpec((D, D), lambda i: (0, 0)),
            pl.BlockSpec((1, D), lambda i: (0, 0)),
            pl.BlockSpec((D, D), lambda i: (0, 0)),
            pl.BlockSpec((1, D), lambda i: (0, 0)),
            pl.BlockSpec((1, 1, ROW_BLK), lambda i: (i, 0, 0)),
            pl.BlockSpec((ROW_BLK, 1), lambda i: (i, 0)),
            pl.BlockSpec((D, 1), lambda i: (0, 0)),
            pl.BlockSpec((1, 1), lambda i: (0, 0)),
            pl.BlockSpec((2 * D, D), lambda i: (0, 0)),
            pl.BlockSpec((1, D), lambda i: (0, 0)),
            pl.BlockSpec((D, D), lambda i: (0, 0)),
            pl.BlockSpec((1, D), lambda i: (0, 0)),
            pl.BlockSpec((D + 16, n_tasks), lambda i: (0, 0)),
            pl.BlockSpec((1, 1), lambda i: (0, 0)),
            pl.BlockSpec((N_GRAPHS, 16), lambda i: (0, 0)),
        ],
        out_specs=pl.BlockSpec((N_GRAPHS, n_tasks), lambda i: (0, 0)),
        out_shape=jax.ShapeDtypeStruct((N_GRAPHS, n_tasks), jnp.float32),
        scratch_shapes=[
            pltpu.VMEM((N_GRAPHS, D), jnp.float32),
            pltpu.VMEM((N_GRAPHS, D), jnp.float32),
        ],
    )(aggpair, h1, W, b, rW, rb, ids3, idc, wa, ba, Wm1, bm1, Wm2, bm2,
      Wc, bc, addin)


def kernel(feats, additional_inputs, W0, b0, rW0, rb0, W1, b1, rW1, rb1,
           w_atom, b_atom, Wm1, bm1, Wm2, bm2, Wc, bc, edge_index, graph_ids):
    pad_src = jnp.zeros((E_PAD,), jnp.int32)
    pad_dst = N_NODES + jnp.arange(E_PAD, dtype=jnp.int32) % (NPAD - N_NODES)
    src3 = jnp.concatenate([edge_index[0], pad_src]).reshape(NW, NCHUNKS, CHUNK)
    dst3 = jnp.concatenate([edge_index[1], pad_dst]).reshape(NW, NCHUNKS, CHUNK)
    b0r = b0.reshape(1, D)
    rb0r = rb0.reshape(1, D)
    b1r = b1.reshape(1, D)
    rb1r = rb1.reshape(1, D)
    bar = b_atom.reshape(1, 1)
    bm1r = bm1.reshape(1, D)
    bm2r = bm2.reshape(1, D)
    bcr = bc.reshape(1, 1)
    ids3 = graph_ids.reshape(GRID_N, 1, ROW_BLK)
    idc = graph_ids.reshape(N_NODES, 1)

    agg0 = _sc_edge_agg(feats, src3, dst3)
    h1 = _gcn_dense(agg0, feats, W0, b0r, rW0, rb0r)
    agg1 = _sc_edge_agg(h1, src3, dst3)
    out = _head(agg1, h1, W1, b1r, rW1, rb1r, ids3, idc, w_atom, bar,
                Wm1, bm1r, Wm2, bm2r, Wc, bcr, additional_inputs)
    return out


# speedup vs baseline: 2.2486x; 2.2486x over previous
import functools

import jax
import jax.numpy as jnp
from jax import lax
from jax.experimental import pallas as pl
from jax.experimental.pallas import tpu as pltpu
from jax.experimental.pallas import tpu_sc as plsc

N_NODES = 10000
N_EDGES = 320000
N_GRAPHS = 64
D = 128

NC = 2
NS = 16
NW = NC * NS

NPAD = 10240
ROWS_PER_TILE = NPAD // NS

CHUNK = 128
NCHUNKS = 80
E_PER_TILE = NCHUNKS * CHUNK
E_PAD = NW * E_PER_TILE - N_EDGES

ROW_BLK = 1000
GRID_N = N_NODES // ROW_BLK

BN_INV = 1.0 / (1.0 + 1e-5) ** 0.5


def _sc_edge_agg(h, src3, dst3):
    mesh = plsc.VectorSubcoreMesh(core_axis_name="c", subcore_axis_name="s")

    @functools.partial(
        pl.kernel,
        mesh=mesh,
        out_type=jax.ShapeDtypeStruct((NC, NPAD, D), jnp.float32),
        scratch_types=[
            pltpu.VMEM((CHUNK,), jnp.int32),
            pltpu.VMEM((CHUNK,), jnp.int32),
            pltpu.VMEM((CHUNK, D), jnp.float32),
            pltpu.VMEM((CHUNK, D), jnp.float32),
            pltpu.VMEM_SHARED((NPAD, D), jnp.float32),
            pltpu.SemaphoreType.DMA,
            pltpu.SemaphoreType.DMA,
            pltpu.SemaphoreType.DMA,
            pltpu.SemaphoreType.DMA,
        ],
    )
    def k(h_hbm, src_hbm, dst_hbm, out_hbm, sidx, didx, rows0, rows1, agg,
          sg0, sg1, ss0, ss1):
        cid = lax.axis_index("c")
        sid = lax.axis_index("s")
        wid = sid * NC + cid
        rows = (rows0, rows1)
        semg = (sg0, sg1)
        sems = (ss0, ss1)

        def zero_row(i, _):
            for j in range(D // 16):
                rows0[i, pl.ds(j * 16, 16)] = jnp.zeros((16,), jnp.float32)
            return 0

        lax.fori_loop(0, CHUNK, zero_row, 0)
        zbase = sid * ROWS_PER_TILE
        for t in range(ROWS_PER_TILE // CHUNK):
            pltpu.sync_copy(rows0, agg.at[pl.ds(zbase + t * CHUNK, CHUNK)])
        plsc.subcore_barrier()

        def body(i, _):
            pltpu.sync_copy(src_hbm.at[wid, i], sidx)
            pltpu.sync_copy(dst_hbm.at[wid, i], didx)
            pltpu.async_copy(h_hbm.at[sidx], rows0, semg[0]).wait()
            pltpu.sync_copy(rows0, agg.at[didx], add=True)
            return 0

        lax.fori_loop(0, NCHUNKS, body, 0)
        plsc.subcore_barrier()

        pltpu.sync_copy(
            agg.at[pl.ds(sid * ROWS_PER_TILE, ROWS_PER_TILE)],
            out_hbm.at[cid, pl.ds(sid * ROWS_PER_TILE, ROWS_PER_TILE)],
        )

    return k(h, src3, dst3)


def _gcn_dense_body(agg_ref, x_ref, W_ref, b_ref, rW_ref, rb_ref, h_ref):
    a = agg_ref[0] + agg_ref[1]
    new = jnp.dot(a, W_ref[...], preferred_element_type=jnp.float32)
    new = jnp.maximum(new + b_ref[...], 0.0)
    r = jnp.dot(x_ref[...], rW_ref[...], preferred_element_type=jnp.float32)
    h_ref[...] = new + jnp.maximum(r + rb_ref[...], 0.0)


def _gcn_dense(aggpair, x, W, b, rW, rb):
    return pl.pallas_call(
        _gcn_dense_body,
        grid=(GRID_N,),
        in_specs=[
            pl.BlockSpec((NC, ROW_BLK, D), lambda i: (0, i, 0)),
            pl.BlockSpec((ROW_BLK, D), lambda i: (i, 0)),
            pl.BlockSpec((D, D), lambda i: (0, 0)),
            pl.BlockSpec((1, D), lambda i: (0, 0)),
            pl.BlockSpec((D, D), lambda i: (0, 0)),
            pl.BlockSpec((1, D), lambda i: (0, 0)),
        ],
        out_specs=pl.BlockSpec((ROW_BLK, D), lambda i: (i, 0)),
        out_shape=jax.ShapeDtypeStruct((N_NODES, D), jnp.float32),
    )(aggpair, x, W, b, rW, rb)


def _head_body(
    agg_ref, h1_ref, W_ref, b_ref, rW_ref, rb_ref, ids_ref, idc_ref,
    wa_ref, ba_ref, Wm1_ref, bm1_ref, Wm2_ref, bm2_ref, Wc_ref, bc_ref,
    add_ref, out_ref, hsum_ref, hmax_ref,
):
    i = pl.program_id(0)
    a = agg_ref[0] + agg_ref[1]
    new = jnp.dot(a, W_ref[...], preferred_element_type=jnp.float32)
    new = jnp.maximum(new + b_ref[...], 0.0)
    r = jnp.dot(h1_ref[...], rW_ref[...], preferred_element_type=jnp.float32)
    h2 = new + jnp.maximum(r + rb_ref[...], 0.0)

    wgt = jax.nn.sigmoid(
        jnp.dot(h2, wa_ref[...], preferred_element_type=jnp.float32)
        + ba_ref[0, 0]
    )

    ids = ids_ref[0, 0, :]
    gi = lax.broadcasted_iota(jnp.int32, (N_GRAPHS, ROW_BLK), 0)
    MT = (gi == ids[None, :]).astype(jnp.float32)
    part_sum = jnp.dot(MT, h2 * wgt, preferred_element_type=jnp.float32)

    @pl.when(i == 0)
    def _():
        hsum_ref[...] = part_sum
        hmax_ref[...] = jnp.full((N_GRAPHS, D), -jnp.inf, jnp.float32)

    @pl.when(i > 0)
    def _():
        hsum_ref[...] += part_sum

    ids_col = idc_ref[...]

    def gmax(g, _):
        hg = jnp.where(ids_col == g, h2, -jnp.inf)
        mx = jnp.max(hg, axis=0)
        hmax_ref[pl.ds(g, 1), :] = jnp.maximum(hmax_ref[pl.ds(g, 1), :], mx[None, :])
        return 0

    lax.fori_loop(jnp.min(ids), jnp.max(ids) + 1, gmax, 0)

    @pl.when(i == GRID_N - 1)
    def _():
        g = jnp.concatenate([hsum_ref[...], hmax_ref[...]], axis=1)
        h1 = jnp.dot(g, Wm1_ref[...], preferred_element_type=jnp.float32)
        h1 = jnp.maximum(h1 + bm1_ref[...], 0.0) * BN_INV
        o = jnp.dot(h1, Wm2_ref[...], preferred_element_type=jnp.float32)
        o = o + bm2_ref[...]
        res = (
            jnp.dot(o, Wc_ref[:D, :], preferred_element_type=jnp.float32)
            + jnp.dot(add_ref[...], Wc_ref[D:, :], preferred_element_type=jnp.float32)
            + bc_ref[0, 0]
        )
        out_ref[...] = res


def _head(aggpair, h1, W, b, rW, rb, ids3, idc, wa, ba, Wm1, bm1, Wm2, bm2,
          Wc, bc, addin):
    n_tasks = Wc.shape[1]
    return pl.pallas_call(
        _head_body,
        grid=(GRID_N,),
        in_specs=[
            pl.BlockSpec((NC, ROW_BLK, D), lambda i: (0, i, 0)),
            pl.BlockSpec((ROW_BLK, D), lambda i: (i, 0)),
            pl.BlockSpec((D, D), lambda i: (0, 0)),
            pl.BlockSpec((1, D), lambda i: (0, 0)),
            pl.BlockSpec((D, D), lambda i: (0, 0)),
            pl.BlockSpec((1, D), lambda i: (0, 0)),
            pl.BlockSpec((1, 1, ROW_BLK), lambda i: (i, 0, 0)),
            pl.BlockSpec((ROW_BLK, 1), lambda i: (i, 0)),
            pl.BlockSpec((D, 1), lambda i: (0, 0)),
            pl.BlockSpec((1, 1), lambda i: (0, 0)),
            pl.BlockSpec((2 * D, D), lambda i: (0, 0)),
            pl.BlockSpec((1, D), lambda i: (0, 0)),
            pl.BlockSpec((D, D), lambda i: (0, 0)),
            pl.BlockSpec((1, D), lambda i: (0, 0)),
            pl.BlockSpec((D + 16, n_tasks), lambda i: (0, 0)),
            pl.BlockSpec((1, 1), lambda i: (0, 0)),
            pl.BlockSpec((N_GRAPHS, 16), lambda i: (0, 0)),
        ],
        out_specs=pl.BlockSpec((N_GRAPHS, n_tasks), lambda i: (0, 0)),
        out_shape=jax.ShapeDtypeStruct((N_GRAPHS, n_tasks), jnp.float32),
        scratch_shapes=[
            pltpu.VMEM((N_GRAPHS, D), jnp.float32),
            pltpu.VMEM((N_GRAPHS, D), jnp.float32),
        ],
    )(aggpair, h1, W, b, rW, rb, ids3, idc, wa, ba, Wm1, bm1, Wm2, bm2,
      Wc, bc, addin)


def kernel(feats, additional_inputs, W0, b0, rW0, rb0, W1, b1, rW1, rb1,
           w_atom, b_atom, Wm1, bm1, Wm2, bm2, Wc, bc, edge_index, graph_ids):
    pad_src = jnp.arange(E_PAD, dtype=jnp.int32) % N_NODES
    pad_dst = N_NODES + jnp.arange(E_PAD, dtype=jnp.int32) % (NPAD - N_NODES)
    src3 = jnp.concatenate([edge_index[0], pad_src]).reshape(NW, NCHUNKS, CHUNK)
    dst3 = jnp.concatenate([edge_index[1], pad_dst]).reshape(NW, NCHUNKS, CHUNK)
    b0r = b0.reshape(1, D)
    rb0r = rb0.reshape(1, D)
    b1r = b1.reshape(1, D)
    rb1r = rb1.reshape(1, D)
    bar = b_atom.reshape(1, 1)
    bm1r = bm1.reshape(1, D)
    bm2r = bm2.reshape(1, D)
    bcr = bc.reshape(1, 1)
    ids3 = graph_ids.reshape(GRID_N, 1, ROW_BLK)
    idc = graph_ids.reshape(N_NODES, 1)

    agg0 = _sc_edge_agg(feats, src3, dst3)
    h1 = _gcn_dense(agg0, feats, W0, b0r, rW0, rb0r)
    agg1 = _sc_edge_agg(h1, src3, dst3)
    out = _head(agg1, h1, W1, b1r, rW1, rb1r, ids3, idc, w_atom, bar,
                Wm1, bm1r, Wm2, bm2r, Wc, bcr, additional_inputs)
    return out


# speedup vs baseline: 3.7952x; 1.6878x over previous
import functools

import jax
import jax.numpy as jnp
from jax import lax
from jax.experimental import pallas as pl
from jax.experimental.pallas import tpu as pltpu
from jax.experimental.pallas import tpu_sc as plsc

N_NODES = 10000
N_EDGES = 320000
N_GRAPHS = 64
D = 128

NC = 2
NS = 16
NW = NC * NS

NPAD = 10240
ROWS_PER_TILE = NPAD // NS

CHUNK = 128
NCHUNKS = 80
E_PER_TILE = NCHUNKS * CHUNK
E_PAD = NW * E_PER_TILE - N_EDGES

ROW_BLK = 1000
GRID_N = N_NODES // ROW_BLK

BN_INV = 1.0 / (1.0 + 1e-5) ** 0.5


def _sc_edge_agg(h, src3, dst3):
    mesh = plsc.VectorSubcoreMesh(core_axis_name="c", subcore_axis_name="s")

    @functools.partial(
        pl.kernel,
        mesh=mesh,
        out_type=jax.ShapeDtypeStruct((NC, NPAD, D), jnp.float32),
        scratch_types=[
            pltpu.VMEM((NCHUNKS // 2, CHUNK), jnp.int32),
            pltpu.VMEM((NCHUNKS // 2, CHUNK), jnp.int32),
            pltpu.VMEM((CHUNK, D), jnp.float32),
            pltpu.VMEM((CHUNK, D), jnp.float32),
            pltpu.VMEM_SHARED((NPAD, D), jnp.float32),
            pltpu.SemaphoreType.DMA,
            pltpu.SemaphoreType.DMA,
            pltpu.SemaphoreType.DMA,
            pltpu.SemaphoreType.DMA,
        ],
    )
    def k(h_hbm, src_hbm, dst_hbm, out_hbm, sidx, didx, rows0, rows1, agg,
          sg0, sg1, ss0, ss1):
        cid = lax.axis_index("c")
        sid = lax.axis_index("s")
        wid = sid * NC + cid
        rows = (rows0, rows1)
        semg = (sg0, sg1)
        sems = (ss0, ss1)

        def zero_row(i, _):
            for j in range(D // 16):
                rows0[i, pl.ds(j * 16, 16)] = jnp.zeros((16,), jnp.float32)
            return 0

        lax.fori_loop(0, CHUNK, zero_row, 0)
        zbase = sid * ROWS_PER_TILE
        for t in range(ROWS_PER_TILE // CHUNK):
            pltpu.sync_copy(rows0, agg.at[pl.ds(zbase + t * CHUNK, CHUNK)])
        plsc.subcore_barrier()

        def pair(i, _):
            for par in range(2):
                c = 2 * i + par
                b = rows[par]
                @pl.when(i > 0)
                def _():
                    pltpu.make_async_copy(
                        b, agg.at[pl.ds(0, CHUNK)], sems[par]
                    ).wait()

                pltpu.async_copy(h_hbm.at[sidx.at[c]], b, semg[par])
                pltpu.make_async_copy(
                    h_hbm.at[pl.ds(0, CHUNK)], b, semg[par]
                ).wait()
                pltpu.async_copy(b, agg.at[didx.at[c]], sems[par], add=True)
            return 0

        for half in range(2):
            pltpu.sync_copy(
                src_hbm.at[wid, pl.ds(half * (NCHUNKS // 2), NCHUNKS // 2)],
                sidx,
            )
            pltpu.sync_copy(
                dst_hbm.at[wid, pl.ds(half * (NCHUNKS // 2), NCHUNKS // 2)],
                didx,
            )
            lax.fori_loop(0, NCHUNKS // 4, pair, 0)
            for par in range(2):
                pltpu.make_async_copy(
                    rows[par], agg.at[pl.ds(0, CHUNK)], sems[par]
                ).wait()
        plsc.subcore_barrier()

        pltpu.sync_copy(
            agg.at[pl.ds(sid * ROWS_PER_TILE, ROWS_PER_TILE)],
            out_hbm.at[cid, pl.ds(sid * ROWS_PER_TILE, ROWS_PER_TILE)],
        )

    return k(h, src3, dst3)


def _gcn_dense_body(agg_ref, x_ref, W_ref, b_ref, rW_ref, rb_ref, h_ref):
    a = agg_ref[0] + agg_ref[1]
    new = jnp.dot(a, W_ref[...], preferred_element_type=jnp.float32)
    new = jnp.maximum(new + b_ref[...], 0.0)
    r = jnp.dot(x_ref[...], rW_ref[...], preferred_element_type=jnp.float32)
    h_ref[...] = new + jnp.maximum(r + rb_ref[...], 0.0)


def _gcn_dense(aggpair, x, W, b, rW, rb):
    return pl.pallas_call(
        _gcn_dense_body,
        grid=(GRID_N,),
        in_specs=[
            pl.BlockSpec((NC, ROW_BLK, D), lambda i: (0, i, 0)),
            pl.BlockSpec((ROW_BLK, D), lambda i: (i, 0)),
            pl.BlockSpec((D, D), lambda i: (0, 0)),
            pl.BlockSpec((1, D), lambda i: (0, 0)),
            pl.BlockSpec((D, D), lambda i: (0, 0)),
            pl.BlockSpec((1, D), lambda i: (0, 0)),
        ],
        out_specs=pl.BlockSpec((ROW_BLK, D), lambda i: (i, 0)),
        out_shape=jax.ShapeDtypeStruct((N_NODES, D), jnp.float32),
    )(aggpair, x, W, b, rW, rb)


def _head_body(
    agg_ref, h1_ref, W_ref, b_ref, rW_ref, rb_ref, ids_ref, idc_ref,
    wa_ref, ba_ref, Wm1_ref, bm1_ref, Wm2_ref, bm2_ref, Wc_ref, bc_ref,
    add_ref, out_ref, hsum_ref, hmax_ref,
):
    i = pl.program_id(0)
    a = agg_ref[0] + agg_ref[1]
    new = jnp.dot(a, W_ref[...], preferred_element_type=jnp.float32)
    new = jnp.maximum(new + b_ref[...], 0.0)
    r = jnp.dot(h1_ref[...], rW_ref[...], preferred_element_type=jnp.float32)
    h2 = new + jnp.maximum(r + rb_ref[...], 0.0)

    wgt = jax.nn.sigmoid(
        jnp.dot(h2, wa_ref[...], preferred_element_type=jnp.float32)
        + ba_ref[0, 0]
    )

    ids = ids_ref[0, 0, :]
    gi = lax.broadcasted_iota(jnp.int32, (N_GRAPHS, ROW_BLK), 0)
    MT = (gi == ids[None, :]).astype(jnp.float32)
    part_sum = jnp.dot(MT, h2 * wgt, preferred_element_type=jnp.float32)

    @pl.when(i == 0)
    def _():
        hsum_ref[...] = part_sum
        hmax_ref[...] = jnp.full((N_GRAPHS, D), -jnp.inf, jnp.float32)

    @pl.when(i > 0)
    def _():
        hsum_ref[...] += part_sum

    ids_col = idc_ref[...]

    def gmax(g, _):
        hg = jnp.where(ids_col == g, h2, -jnp.inf)
        mx = jnp.max(hg, axis=0)
        hmax_ref[pl.ds(g, 1), :] = jnp.maximum(hmax_ref[pl.ds(g, 1), :], mx[None, :])
        return 0

    lax.fori_loop(jnp.min(ids), jnp.max(ids) + 1, gmax, 0)

    @pl.when(i == GRID_N - 1)
    def _():
        g = jnp.concatenate([hsum_ref[...], hmax_ref[...]], axis=1)
        h1 = jnp.dot(g, Wm1_ref[...], preferred_element_type=jnp.float32)
        h1 = jnp.maximum(h1 + bm1_ref[...], 0.0) * BN_INV
        o = jnp.dot(h1, Wm2_ref[...], preferred_element_type=jnp.float32)
        o = o + bm2_ref[...]
        res = (
            jnp.dot(o, Wc_ref[:D, :], preferred_element_type=jnp.float32)
            + jnp.dot(add_ref[...], Wc_ref[D:, :], preferred_element_type=jnp.float32)
            + bc_ref[0, 0]
        )
        out_ref[...] = res


def _head(aggpair, h1, W, b, rW, rb, ids3, idc, wa, ba, Wm1, bm1, Wm2, bm2,
          Wc, bc, addin):
    n_tasks = Wc.shape[1]
    return pl.pallas_call(
        _head_body,
        grid=(GRID_N,),
        in_specs=[
            pl.BlockSpec((NC, ROW_BLK, D), lambda i: (0, i, 0)),
            pl.BlockSpec((ROW_BLK, D), lambda i: (i, 0)),
            pl.BlockSpec((D, D), lambda i: (0, 0)),
            pl.BlockSpec((1, D), lambda i: (0, 0)),
            pl.BlockSpec((D, D), lambda i: (0, 0)),
            pl.BlockSpec((1, D), lambda i: (0, 0)),
            pl.BlockSpec((1, 1, ROW_BLK), lambda i: (i, 0, 0)),
            pl.BlockSpec((ROW_BLK, 1), lambda i: (i, 0)),
            pl.BlockSpec((D, 1), lambda i: (0, 0)),
            pl.BlockSpec((1, 1), lambda i: (0, 0)),
            pl.BlockSpec((2 * D, D), lambda i: (0, 0)),
            pl.BlockSpec((1, D), lambda i: (0, 0)),
            pl.BlockSpec((D, D), lambda i: (0, 0)),
            pl.BlockSpec((1, D), lambda i: (0, 0)),
            pl.BlockSpec((D + 16, n_tasks), lambda i: (0, 0)),
            pl.BlockSpec((1, 1), lambda i: (0, 0)),
            pl.BlockSpec((N_GRAPHS, 16), lambda i: (0, 0)),
        ],
        out_specs=pl.BlockSpec((N_GRAPHS, n_tasks), lambda i: (0, 0)),
        out_shape=jax.ShapeDtypeStruct((N_GRAPHS, n_tasks), jnp.float32),
        scratch_shapes=[
            pltpu.VMEM((N_GRAPHS, D), jnp.float32),
            pltpu.VMEM((N_GRAPHS, D), jnp.float32),
        ],
    )(aggpair, h1, W, b, rW, rb, ids3, idc, wa, ba, Wm1, bm1, Wm2, bm2,
      Wc, bc, addin)


def kernel(feats, additional_inputs, W0, b0, rW0, rb0, W1, b1, rW1, rb1,
           w_atom, b_atom, Wm1, bm1, Wm2, bm2, Wc, bc, edge_index, graph_ids):
    pad_src = jnp.arange(E_PAD, dtype=jnp.int32) % N_NODES
    pad_dst = N_NODES + jnp.arange(E_PAD, dtype=jnp.int32) % (NPAD - N_NODES)
    src3 = jnp.concatenate([edge_index[0], pad_src]).reshape(NW, NCHUNKS, CHUNK)
    dst3 = jnp.concatenate([edge_index[1], pad_dst]).reshape(NW, NCHUNKS, CHUNK)
    b0r = b0.reshape(1, D)
    rb0r = rb0.reshape(1, D)
    b1r = b1.reshape(1, D)
    rb1r = rb1.reshape(1, D)
    bar = b_atom.reshape(1, 1)
    bm1r = bm1.reshape(1, D)
    bm2r = bm2.reshape(1, D)
    bcr = bc.reshape(1, 1)
    ids3 = graph_ids.reshape(GRID_N, 1, ROW_BLK)
    idc = graph_ids.reshape(N_NODES, 1)

    agg0 = _sc_edge_agg(feats, src3, dst3)
    h1 = _gcn_dense(agg0, feats, W0, b0r, rW0, rb0r)
    agg1 = _sc_edge_agg(h1, src3, dst3)
    out = _head(agg1, h1, W1, b1r, rW1, rb1r, ids3, idc, w_atom, bar,
                Wm1, bm1r, Wm2, bm2r, Wc, bcr, additional_inputs)
    return out
